# baseline (speedup 1.0000x reference)
import functools

import jax
import jax.numpy as jnp
from jax import lax
from jax.experimental import pallas as pl
from jax.experimental.pallas import tpu as pltpu
from jax.experimental.pallas import tpu_sc as plsc

H = 768
L = 16
NG = H // L
TOK = 4 * 4096
NW = 32
PER_W = TOK // NW
C = 16
NCHUNK = PER_W // C
EPS = 1e-12


def _lane_sum(x):
    for sh in (1, 2, 4, 8):
        perm = lax.iota(jnp.int32, L) ^ sh
        x = x + x.at[perm].get(mode="promise_in_bounds")
    return x


def _rsqrt(v):
    i = lax.bitcast_convert_type(v, jnp.int32)
    i = jnp.int32(0x5F3759DF) - lax.shift_right_arithmetic(i, 1)
    y = lax.bitcast_convert_type(i, jnp.float32)
    for _ in range(3):
        y = y * (1.5 - 0.5 * v * y * y)
    return y


def _body(ids_w, ids_e, ids_t, word_emb, entity_emb, triple_emb, pos_emb,
          gamma, beta, out,
          idxw_v, idxe_v, idxt_v, gamma_v, beta_v,
          bw0, be0, bt0, bp0, bw1, be1, bt1, bp1, ob0, ob1,
          sem_g0, sem_g1, sem_o0, sem_o1):
    wid = lax.axis_index("s") * 2 + lax.axis_index("c")
    base = wid * PER_W

    pltpu.sync_copy(ids_w.at[pl.ds(base, PER_W)], idxw_v)
    pltpu.sync_copy(ids_e.at[pl.ds(base, PER_W)], idxe_v)
    pltpu.sync_copy(ids_t.at[pl.ds(base, PER_W)], idxt_v)
    pltpu.sync_copy(gamma, gamma_v)
    pltpu.sync_copy(beta, beta_v)

    bufs = ((bw0, be0, bt0, bp0), (bw1, be1, bt1, bp1))
    obufs = (ob0, ob1)
    gsems = (sem_g0, sem_g1)
    osems = (sem_o0, sem_o1)

    def fire(c, s):
        iw = idxw_v[pl.ds(c * C, C)]
        ie = idxe_v[pl.ds(c * C, C)]
        it = idxt_v[pl.ds(c * C, C)]
        bw, be, bt, bp = bufs[s]
        pltpu.async_copy(word_emb.at[iw], bw, gsems[s])
        pltpu.async_copy(entity_emb.at[ie], be, gsems[s])
        pltpu.async_copy(triple_emb.at[it], bt, gsems[s])
        pltpu.async_copy(pos_emb.at[it], bp, gsems[s])

    def compute(s):
        bw, be, bt, bp = bufs[s]
        ob = obufs[s]

        def token(t, _):
            acc = jnp.zeros((L,), jnp.float32)
            acc2 = jnp.zeros((L,), jnp.float32)
            for g in range(NG):
                sl = pl.ds(g * L, L)
                a = (bw[t, sl] + be[t, sl]) + (bt[t, sl] + bp[t, sl])
                ob[t, sl] = a
                acc = acc + a
                acc2 = acc2 + a * a
            mu = _lane_sum(acc) * (1.0 / H)
            ex2 = _lane_sum(acc2) * (1.0 / H)
            r = _rsqrt(ex2 - mu * mu + EPS)
            for g in range(NG):
                sl = pl.ds(g * L, L)
                x = ob[t, sl]
                ob[t, sl] = (x - mu) * r * gamma_v[sl] + beta_v[sl]
            return 0

        lax.fori_loop(0, C, token, 0)

    def wait_gathers(s):
        bw, be, bt, bp = bufs[s]
        iw = idxw_v[pl.ds(0, C)]
        pltpu.make_async_copy(word_emb.at[iw], bw, gsems[s]).wait()
        pltpu.make_async_copy(word_emb.at[iw], be, gsems[s]).wait()
        pltpu.make_async_copy(word_emb.at[iw], bt, gsems[s]).wait()
        pltpu.make_async_copy(word_emb.at[iw], bp, gsems[s]).wait()

    fire(0, 0)

    def pair(k, _):
        for s in (0, 1):
            c = 2 * k + s

            @pl.when(c + 1 < NCHUNK)
            def _():
                fire(c + 1, 1 - s)

            wait_gathers(s)

            @pl.when(c >= 2)
            def _():
                pltpu.make_async_copy(
                    obufs[s], out.at[pl.ds(base, C)], osems[s]).wait()

            compute(s)
            pltpu.async_copy(obufs[s], out.at[pl.ds(base + c * C, C)], osems[s])
        return 0

    lax.fori_loop(0, NCHUNK // 2, pair, 0)

    pltpu.make_async_copy(ob0, out.at[pl.ds(base, C)], sem_o0).wait()
    pltpu.make_async_copy(ob1, out.at[pl.ds(base, C)], sem_o1).wait()


def kernel(input_ids, entity_ids, triple_ids, position_ids,
           word_emb, entity_emb, triple_emb, pos_emb, gamma, beta):
    del position_ids
    ids_w = input_ids.reshape(-1).astype(jnp.int32)
    ids_e = entity_ids.reshape(-1).astype(jnp.int32)
    ids_t = triple_ids.reshape(-1).astype(jnp.int32)

    mesh = plsc.VectorSubcoreMesh(core_axis_name="c", subcore_axis_name="s")
    run = functools.partial(
        pl.kernel, mesh=mesh,
        out_type=jax.ShapeDtypeStruct((TOK, H), jnp.float32),
        scratch_types=[
            pltpu.VMEM((PER_W,), jnp.int32),
            pltpu.VMEM((PER_W,), jnp.int32),
            pltpu.VMEM((PER_W,), jnp.int32),
            pltpu.VMEM((H,), jnp.float32),
            pltpu.VMEM((H,), jnp.float32),
        ] + [pltpu.VMEM((C, H), jnp.float32)] * 10 + [
            pltpu.SemaphoreType.DMA,
            pltpu.SemaphoreType.DMA,
            pltpu.SemaphoreType.DMA,
            pltpu.SemaphoreType.DMA,
        ],
    )(_body)
    out = run(ids_w, ids_e, ids_t, word_emb, entity_emb, triple_emb, pos_emb,
              gamma, beta)
    return out.reshape(4, 4096, H)


# speedup vs baseline: 1.3540x; 1.3540x over previous
import functools

import jax
import jax.numpy as jnp
from jax import lax
from jax.experimental import pallas as pl
from jax.experimental.pallas import tpu as pltpu
from jax.experimental.pallas import tpu_sc as plsc

H = 768
L = 16
NG = H // L
TOK = 4 * 4096
NW = 32
PER_W = TOK // NW
C = 16
NCHUNK = PER_W // C
EPS = 1e-12


def _lane_sum(x):
    for sh in (1, 2, 4, 8):
        perm = lax.iota(jnp.int32, L) ^ sh
        x = x + x.at[perm].get(mode="promise_in_bounds")
    return x


def _rsqrt(v):
    i = lax.bitcast_convert_type(v, jnp.int32)
    i = jnp.int32(0x5F3759DF) - lax.shift_right_arithmetic(i, 1)
    y = lax.bitcast_convert_type(i, jnp.float32)
    for _ in range(3):
        y = y * (1.5 - 0.5 * v * y * y)
    return y


def _body(ids_w, ids_e, ids_t, word_emb, entity_emb, triple_emb, pos_emb,
          gamma, beta, out,
          idxw_v, idxe_v, idxt_v, gamma_v, beta_v,
          bw0, be0, bt0, bp0, bw1, be1, bt1, bp1, ob0, ob1, pq_v,
          sem_g0, sem_g1, sem_o0, sem_o1):
    wid = lax.axis_index("s") * 2 + lax.axis_index("c")
    base = wid * PER_W

    pltpu.sync_copy(ids_w.at[pl.ds(base, PER_W)], idxw_v)
    pltpu.sync_copy(ids_e.at[pl.ds(base, PER_W)], idxe_v)
    pltpu.sync_copy(ids_t.at[pl.ds(base, PER_W)], idxt_v)
    pltpu.sync_copy(gamma, gamma_v)
    pltpu.sync_copy(beta, beta_v)

    bufs = ((bw0, be0, bt0, bp0), (bw1, be1, bt1, bp1))
    obufs = (ob0, ob1)
    gsems = (sem_g0, sem_g1)
    osems = (sem_o0, sem_o1)

    def fire(c, s):
        iw = idxw_v[pl.ds(c * C, C)]
        ie = idxe_v[pl.ds(c * C, C)]
        it = idxt_v[pl.ds(c * C, C)]
        bw, be, bt, bp = bufs[s]
        pltpu.async_copy(word_emb.at[iw], bw, gsems[s])
        pltpu.async_copy(entity_emb.at[ie], be, gsems[s])
        pltpu.async_copy(triple_emb.at[it], bt, gsems[s])
        pltpu.async_copy(pos_emb.at[it], bp, gsems[s])

    def compute(s):
        bw, be, bt, bp = bufs[s]
        ob = obufs[s]

        def token(t, _):
            accs = [jnp.zeros((L,), jnp.float32) for _ in range(4)]
            acc2s = [jnp.zeros((L,), jnp.float32) for _ in range(4)]
            for g in range(NG):
                sl = pl.ds(g * L, L)
                a = (bw[t, sl] + be[t, sl]) + (bt[t, sl] + bp[t, sl])
                ob[t, sl] = a
                accs[g % 4] = accs[g % 4] + a
                acc2s[g % 4] = acc2s[g % 4] + a * a
            acc = (accs[0] + accs[1]) + (accs[2] + accs[3])
            acc2 = (acc2s[0] + acc2s[1]) + (acc2s[2] + acc2s[3])
            mu = _lane_sum(acc) * (1.0 / H)
            ex2 = _lane_sum(acc2) * (1.0 / H)
            r = _rsqrt(ex2 - mu * mu + EPS)
            pq_v[t, pl.ds(0, L)] = r
            pq_v[t, pl.ds(L, L)] = -mu * r
            return 0

        lax.fori_loop(0, C, token, 0)

        ps = tuple(pq_v[t, pl.ds(0, L)] for t in range(C))
        qs = tuple(pq_v[t, pl.ds(L, L)] for t in range(C))

        def grp(g, carry):
            cp, cq = carry
            sl = pl.ds(g * L, L)
            gv = gamma_v[sl]
            bv = beta_v[sl]
            for t in range(C):
                x = ob[t, sl]
                ob[t, sl] = (x * cp[t] + cq[t]) * gv + bv
            return carry

        lax.fori_loop(0, NG, grp, (ps, qs))

    def wait_gathers(s):
        bw, be, bt, bp = bufs[s]
        iw = idxw_v[pl.ds(0, C)]
        pltpu.make_async_copy(word_emb.at[iw], bw, gsems[s]).wait()
        pltpu.make_async_copy(word_emb.at[iw], be, gsems[s]).wait()
        pltpu.make_async_copy(word_emb.at[iw], bt, gsems[s]).wait()
        pltpu.make_async_copy(word_emb.at[iw], bp, gsems[s]).wait()

    fire(0, 0)

    def pair(k, _):
        for s in (0, 1):
            c = 2 * k + s

            @pl.when(c + 1 < NCHUNK)
            def _():
                fire(c + 1, 1 - s)

            wait_gathers(s)

            @pl.when(c >= 2)
            def _():
                pltpu.make_async_copy(
                    obufs[s], out.at[pl.ds(base, C)], osems[s]).wait()

            compute(s)
            pltpu.async_copy(obufs[s], out.at[pl.ds(base + c * C, C)], osems[s])
        return 0

    lax.fori_loop(0, NCHUNK // 2, pair, 0)

    pltpu.make_async_copy(ob0, out.at[pl.ds(base, C)], sem_o0).wait()
    pltpu.make_async_copy(ob1, out.at[pl.ds(base, C)], sem_o1).wait()


def kernel(input_ids, entity_ids, triple_ids, position_ids,
           word_emb, entity_emb, triple_emb, pos_emb, gamma, beta):
    del position_ids
    ids_w = input_ids.reshape(-1).astype(jnp.int32)
    ids_e = entity_ids.reshape(-1).astype(jnp.int32)
    ids_t = triple_ids.reshape(-1).astype(jnp.int32)

    mesh = plsc.VectorSubcoreMesh(core_axis_name="c", subcore_axis_name="s")
    run = functools.partial(
        pl.kernel, mesh=mesh,
        out_type=jax.ShapeDtypeStruct((TOK, H), jnp.float32),
        scratch_types=[
            pltpu.VMEM((PER_W,), jnp.int32),
            pltpu.VMEM((PER_W,), jnp.int32),
            pltpu.VMEM((PER_W,), jnp.int32),
            pltpu.VMEM((H,), jnp.float32),
            pltpu.VMEM((H,), jnp.float32),
        ] + [pltpu.VMEM((C, H), jnp.float32)] * 10 + [
            pltpu.VMEM((C, 2 * L), jnp.float32),
            pltpu.SemaphoreType.DMA,
            pltpu.SemaphoreType.DMA,
            pltpu.SemaphoreType.DMA,
            pltpu.SemaphoreType.DMA,
        ],
    )(_body)
    out = run(ids_w, ids_e, ids_t, word_emb, entity_emb, triple_emb, pos_emb,
              gamma, beta)
    return out.reshape(4, 4096, H)


# speedup vs baseline: 1.6460x; 1.2157x over previous
import functools

import jax
import jax.numpy as jnp
from jax import lax
from jax.experimental import pallas as pl
from jax.experimental.pallas import tpu as pltpu
from jax.experimental.pallas import tpu_sc as plsc

H = 768
L = 16
NG = H // L
TOK = 4 * 4096
NW = 32
PER_W = TOK // NW
C = 16
NCHUNK = PER_W // C
EPS = 1e-12


def _lane_sum(x):
    for sh in (1, 2, 4, 8):
        perm = lax.iota(jnp.int32, L) ^ sh
        x = x + x.at[perm].get(mode="promise_in_bounds")
    return x


def _rsqrt(v):
    i = lax.bitcast_convert_type(v, jnp.int32)
    i = jnp.int32(0x5F3759DF) - lax.shift_right_arithmetic(i, 1)
    y = lax.bitcast_convert_type(i, jnp.float32)
    for _ in range(2):
        y = y * (1.5 - 0.5 * v * y * y)
    return y


def _body(ids_w, ids_e, ids_t, word_emb, entity_emb, triple_emb, pos_emb,
          gamma, beta, out,
          idxw_v, idxe_v, idxt_v, gamma_v, beta_v,
          bw0, be0, bt0, bp0, bw1, be1, bt1, bp1, ob0, ob1, pq_v,
          sem_g0, sem_g1, sem_o0, sem_o1):
    wid = lax.axis_index("s") * 2 + lax.axis_index("c")
    base = wid * PER_W

    pltpu.sync_copy(ids_w.at[pl.ds(base, PER_W)], idxw_v)
    pltpu.sync_copy(ids_e.at[pl.ds(base, PER_W)], idxe_v)
    pltpu.sync_copy(ids_t.at[pl.ds(base, PER_W)], idxt_v)
    pltpu.sync_copy(gamma, gamma_v)
    pltpu.sync_copy(beta, beta_v)

    bufs = ((bw0, be0, bt0, bp0), (bw1, be1, bt1, bp1))
    obufs = (ob0, ob1)
    gsems = (sem_g0, sem_g1)
    osems = (sem_o0, sem_o1)

    def fire(c, s):
        iw = idxw_v[pl.ds(c * C, C)]
        ie = idxe_v[pl.ds(c * C, C)]
        it = idxt_v[pl.ds(c * C, C)]
        bw, be, bt, bp = bufs[s]
        pltpu.async_copy(word_emb.at[iw], bw, gsems[s])
        pltpu.async_copy(entity_emb.at[ie], be, gsems[s])
        pltpu.async_copy(triple_emb.at[it], bt, gsems[s])
        pltpu.async_copy(pos_emb.at[it], bp, gsems[s])

    def compute(s):
        bw, be, bt, bp = bufs[s]
        ob = obufs[s]

        def tokpair(i, _):
            for t in (2 * i, 2 * i + 1):
                accs = [jnp.zeros((L,), jnp.float32) for _ in range(4)]
                acc2s = [jnp.zeros((L,), jnp.float32) for _ in range(4)]
                for g in range(NG):
                    sl = pl.ds(g * L, L)
                    a = (bw[t, sl] + be[t, sl]) + (bt[t, sl] + bp[t, sl])
                    ob[t, sl] = a
                    accs[g % 4] = accs[g % 4] + a
                    acc2s[g % 4] = acc2s[g % 4] + a * a
                acc = (accs[0] + accs[1]) + (accs[2] + accs[3])
                acc2 = (acc2s[0] + acc2s[1]) + (acc2s[2] + acc2s[3])
                mu = _lane_sum(acc) * (1.0 / H)
                ex2 = _lane_sum(acc2) * (1.0 / H)
                r = _rsqrt(ex2 - mu * mu + EPS)
                pq_v[t, pl.ds(0, L)] = r
                pq_v[t, pl.ds(L, L)] = -mu * r
            return 0

        lax.fori_loop(0, C // 2, tokpair, 0)

        for half in (0, 1):
            t0 = half * (C // 2)
            ps = tuple(pq_v[t0 + t, pl.ds(0, L)] for t in range(C // 2))
            qs = tuple(pq_v[t0 + t, pl.ds(L, L)] for t in range(C // 2))

            def grp(g, carry, t0=t0):
                cp, cq = carry
                sl = pl.ds(g * L, L)
                gv = gamma_v[sl]
                bv = beta_v[sl]
                for t in range(C // 2):
                    x = ob[t0 + t, sl]
                    ob[t0 + t, sl] = (x * cp[t] + cq[t]) * gv + bv
                return carry

            lax.fori_loop(0, NG, grp, (ps, qs))

    def wait_gathers(s):
        bw, be, bt, bp = bufs[s]
        iw = idxw_v[pl.ds(0, C)]
        pltpu.make_async_copy(word_emb.at[iw], bw, gsems[s]).wait()
        pltpu.make_async_copy(word_emb.at[iw], be, gsems[s]).wait()
        pltpu.make_async_copy(word_emb.at[iw], bt, gsems[s]).wait()
        pltpu.make_async_copy(word_emb.at[iw], bp, gsems[s]).wait()

    fire(0, 0)

    def pair(k, _):
        for s in (0, 1):
            c = 2 * k + s

            @pl.when(c + 1 < NCHUNK)
            def _():
                fire(c + 1, 1 - s)

            wait_gathers(s)

            @pl.when(c >= 2)
            def _():
                pltpu.make_async_copy(
                    obufs[s], out.at[pl.ds(base, C)], osems[s]).wait()

            compute(s)
            pltpu.async_copy(obufs[s], out.at[pl.ds(base + c * C, C)], osems[s])
        return 0

    lax.fori_loop(0, NCHUNK // 2, pair, 0)

    pltpu.make_async_copy(ob0, out.at[pl.ds(base, C)], sem_o0).wait()
    pltpu.make_async_copy(ob1, out.at[pl.ds(base, C)], sem_o1).wait()


def kernel(input_ids, entity_ids, triple_ids, position_ids,
           word_emb, entity_emb, triple_emb, pos_emb, gamma, beta):
    del position_ids
    ids_w = input_ids.reshape(-1).astype(jnp.int32)
    ids_e = entity_ids.reshape(-1).astype(jnp.int32)
    ids_t = triple_ids.reshape(-1).astype(jnp.int32)

    mesh = plsc.VectorSubcoreMesh(core_axis_name="c", subcore_axis_name="s")
    run = functools.partial(
        pl.kernel, mesh=mesh,
        out_type=jax.ShapeDtypeStruct((TOK, H), jnp.float32),
        scratch_types=[
            pltpu.VMEM((PER_W,), jnp.int32),
            pltpu.VMEM((PER_W,), jnp.int32),
            pltpu.VMEM((PER_W,), jnp.int32),
            pltpu.VMEM((H,), jnp.float32),
            pltpu.VMEM((H,), jnp.float32),
        ] + [pltpu.VMEM((C, H), jnp.float32)] * 10 + [
            pltpu.VMEM((C, 2 * L), jnp.float32),
            pltpu.SemaphoreType.DMA,
            pltpu.SemaphoreType.DMA,
            pltpu.SemaphoreType.DMA,
            pltpu.SemaphoreType.DMA,
        ],
    )(_body)
    out = run(ids_w, ids_e, ids_t, word_emb, entity_emb, triple_emb, pos_emb,
              gamma, beta)
    return out.reshape(4, 4096, H)


# speedup vs baseline: 1.8610x; 1.1306x over previous
import functools

import jax
import jax.numpy as jnp
from jax import lax
from jax.experimental import pallas as pl
from jax.experimental.pallas import tpu as pltpu
from jax.experimental.pallas import tpu_sc as plsc

H = 768
L = 16
NG = H // L
TOK = 4 * 4096
NW = 32
PER_W = TOK // NW
C = 16
NCHUNK = PER_W // C
EPS = 1e-12


def _lane_sum(x):
    for sh in (1, 2, 4, 8):
        perm = lax.iota(jnp.int32, L) ^ sh
        x = x + x.at[perm].get(mode="promise_in_bounds")
    return x


def _rsqrt(v):
    i = lax.bitcast_convert_type(v, jnp.int32)
    i = jnp.int32(0x5F3759DF) - lax.shift_right_arithmetic(i, 1)
    y = lax.bitcast_convert_type(i, jnp.float32)
    for _ in range(2):
        y = y * (1.5 - 0.5 * v * y * y)
    return y


def _body(ids_w, ids_e, ids_t, word_emb, entity_emb, triple_emb, pos_emb,
          gamma, beta, out,
          idxw_v, idxe_v, idxt_v, gamma_v, beta_v,
          bw0, be0, bt0, bp0, bw1, be1, bt1, bp1, ob0, ob1, pq_v,
          sem_g0, sem_g1, sem_o0, sem_o1):
    wid = lax.axis_index("s") * 2 + lax.axis_index("c")
    base = wid * PER_W

    pltpu.sync_copy(ids_w.at[pl.ds(base, PER_W)], idxw_v)
    pltpu.sync_copy(ids_e.at[pl.ds(base, PER_W)], idxe_v)
    pltpu.sync_copy(ids_t.at[pl.ds(base, PER_W)], idxt_v)
    pltpu.sync_copy(gamma, gamma_v)
    pltpu.sync_copy(beta, beta_v)

    bufs = ((bw0, be0, bt0, bp0), (bw1, be1, bt1, bp1))
    obufs = (ob0, ob1)
    gsems = (sem_g0, sem_g1)
    osems = (sem_o0, sem_o1)

    def fire(c, s):
        iw = idxw_v[pl.ds(c * C, C)]
        ie = idxe_v[pl.ds(c * C, C)]
        it = idxt_v[pl.ds(c * C, C)]
        bw, be, bt, bp = bufs[s]
        pltpu.async_copy(word_emb.at[iw], bw, gsems[s])
        pltpu.async_copy(entity_emb.at[ie], be, gsems[s])
        pltpu.async_copy(triple_emb.at[it], bt, gsems[s])
        pltpu.async_copy(pos_emb.at[it], bp, gsems[s])

    def compute(s):
        bw, be, bt, bp = bufs[s]
        ob = obufs[s]

        @plsc.parallel_loop(0, C // 2, 1)
        def tokpair(i):
            for t in (2 * i, 2 * i + 1):
                accs = [jnp.zeros((L,), jnp.float32) for _ in range(4)]
                acc2s = [jnp.zeros((L,), jnp.float32) for _ in range(4)]
                for g in range(NG):
                    sl = pl.ds(g * L, L)
                    a = (bw[t, sl] + be[t, sl]) + (bt[t, sl] + bp[t, sl])
                    ob[t, sl] = a
                    accs[g % 4] = accs[g % 4] + a
                    acc2s[g % 4] = acc2s[g % 4] + a * a
                acc = (accs[0] + accs[1]) + (accs[2] + accs[3])
                acc2 = (acc2s[0] + acc2s[1]) + (acc2s[2] + acc2s[3])
                mu = _lane_sum(acc) * (1.0 / H)
                ex2 = _lane_sum(acc2) * (1.0 / H)
                r = _rsqrt(ex2 - mu * mu + EPS)
                pq_v[t, pl.ds(0, L)] = r
                pq_v[t, pl.ds(L, L)] = -mu * r

        for half in (0, 1):
            t0 = half * (C // 2)
            ps = tuple(pq_v[t0 + t, pl.ds(0, L)] for t in range(C // 2))
            qs = tuple(pq_v[t0 + t, pl.ds(L, L)] for t in range(C // 2))

            @plsc.parallel_loop(0, NG, 1, unroll=2, carry=(ps, qs))
            def grp(g, carry, t0=t0):
                cp, cq = carry
                sl = pl.ds(g * L, L)
                gv = gamma_v[sl]
                bv = beta_v[sl]
                for t in range(C // 2):
                    x = ob[t0 + t, sl]
                    ob[t0 + t, sl] = (x * cp[t] + cq[t]) * gv + bv
                return carry

    def wait_gathers(s):
        bw, be, bt, bp = bufs[s]
        iw = idxw_v[pl.ds(0, C)]
        pltpu.make_async_copy(word_emb.at[iw], bw, gsems[s]).wait()
        pltpu.make_async_copy(word_emb.at[iw], be, gsems[s]).wait()
        pltpu.make_async_copy(word_emb.at[iw], bt, gsems[s]).wait()
        pltpu.make_async_copy(word_emb.at[iw], bp, gsems[s]).wait()

    fire(0, 0)

    def pair(k, _):
        for s in (0, 1):
            c = 2 * k + s

            @pl.when(c + 1 < NCHUNK)
            def _():
                fire(c + 1, 1 - s)

            wait_gathers(s)

            @pl.when(c >= 2)
            def _():
                pltpu.make_async_copy(
                    obufs[s], out.at[pl.ds(base, C)], osems[s]).wait()

            compute(s)
            pltpu.async_copy(obufs[s], out.at[pl.ds(base + c * C, C)], osems[s])
        return 0

    lax.fori_loop(0, NCHUNK // 2, pair, 0)

    pltpu.make_async_copy(ob0, out.at[pl.ds(base, C)], sem_o0).wait()
    pltpu.make_async_copy(ob1, out.at[pl.ds(base, C)], sem_o1).wait()


def kernel(input_ids, entity_ids, triple_ids, position_ids,
           word_emb, entity_emb, triple_emb, pos_emb, gamma, beta):
    del position_ids
    ids_w = input_ids.reshape(-1).astype(jnp.int32)
    ids_e = entity_ids.reshape(-1).astype(jnp.int32)
    ids_t = triple_ids.reshape(-1).astype(jnp.int32)

    mesh = plsc.VectorSubcoreMesh(core_axis_name="c", subcore_axis_name="s")
    run = functools.partial(
        pl.kernel, mesh=mesh,
        out_type=jax.ShapeDtypeStruct((TOK, H), jnp.float32),
        scratch_types=[
            pltpu.VMEM((PER_W,), jnp.int32),
            pltpu.VMEM((PER_W,), jnp.int32),
            pltpu.VMEM((PER_W,), jnp.int32),
            pltpu.VMEM((H,), jnp.float32),
            pltpu.VMEM((H,), jnp.float32),
        ] + [pltpu.VMEM((C, H), jnp.float32)] * 10 + [
            pltpu.VMEM((C, 2 * L), jnp.float32),
            pltpu.SemaphoreType.DMA,
            pltpu.SemaphoreType.DMA,
            pltpu.SemaphoreType.DMA,
            pltpu.SemaphoreType.DMA,
        ],
    )(_body)
    out = run(ids_w, ids_e, ids_t, word_emb, entity_emb, triple_emb, pos_emb,
              gamma, beta)
    return out.reshape(4, 4096, H)
